# baseline (speedup 1.0000x reference)
import functools

import jax
import jax.numpy as jnp
import numpy as np
from jax import lax
from jax.experimental import pallas as pl
from jax.experimental.pallas import tpu as pltpu
from jax.experimental.pallas import tpu_sc as plsc

T, D, H, DH = 2048, 2048, 16, 128
M = 4096
WINDOW = 256
TOPK = 8
TB = 256
NW = 32
TOK_PER_W = T // NW
CH = 4
NCH = TOK_PER_W // CH
ROWS = CH * TOPK


def _proj_body(h_ref, wq_ref, wk_ref, wv_ref, wkt_ref, q_ref, k_ref, v_ref,
               hwk_ref):
    h = h_ref[...].astype(jnp.bfloat16)
    q_ref[...] = jnp.dot(h, wq_ref[...].astype(jnp.bfloat16),
                         preferred_element_type=jnp.float32)
    k_ref[...] = jnp.dot(h, wk_ref[...].astype(jnp.bfloat16),
                         preferred_element_type=jnp.float32)
    v_ref[...] = jnp.dot(h, wv_ref[...].astype(jnp.bfloat16),
                         preferred_element_type=jnp.float32)
    hwk_ref[...] = lax.dot_general(h, wkt_ref[...].astype(jnp.bfloat16),
                                   (((1,), (1,)), ((), ())),
                                   preferred_element_type=jnp.float32)


def _proj(h, Wq, Wk, Wv):
    grid = (T // TB, D // TB)
    out = jax.ShapeDtypeStruct((T, D), jnp.float32)
    return pl.pallas_call(
        _proj_body,
        grid=grid,
        in_specs=[
            pl.BlockSpec((TB, D), lambda i, j: (i, 0)),
            pl.BlockSpec((D, TB), lambda i, j: (0, j)),
            pl.BlockSpec((D, TB), lambda i, j: (0, j)),
            pl.BlockSpec((D, TB), lambda i, j: (0, j)),
            pl.BlockSpec((TB, D), lambda i, j: (j, 0)),
        ],
        out_specs=[pl.BlockSpec((TB, TB), lambda i, j: (i, j))] * 4,
        out_shape=[out, out, out, out],
    )(h, Wq, Wk, Wv, Wk)


TB2 = 128


def _topk_body(h_ref, hwk_ref, mem_ref, idx_ref, mw_ref):
    h = h_ref[...].astype(jnp.bfloat16)
    mem = mem_ref[...].astype(jnp.bfloat16)
    s = lax.dot_general(h, mem, (((1,), (1,)), ((), ())),
                        preferred_element_type=jnp.float32)
    ml = lax.dot_general(hwk_ref[...].astype(jnp.bfloat16), mem,
                         (((1,), (1,)), ((), ())),
                         preferred_element_type=jnp.float32)
    colid = lax.broadcasted_iota(jnp.int32, (TB2, M), 1)
    mls = []
    for i in range(TOPK):
        mx = jnp.max(s, axis=1, keepdims=True)
        amin = jnp.min(jnp.where(s >= mx, colid, M), axis=1, keepdims=True)
        onehot = colid == amin
        mls.append(jnp.sum(jnp.where(onehot, ml, 0.0), axis=1))
        idx_ref[i, :] = amin[:, 0]
        s = jnp.where(onehot, -jnp.inf, s)
    mlk = jnp.stack(mls, axis=0) * np.float32(1.0 / np.sqrt(D))
    z = mlk - jnp.max(mlk, axis=0, keepdims=True)
    e = jnp.exp(z)
    mw_ref[...] = e / jnp.sum(e, axis=0, keepdims=True)


def _topk(h, hwk, memory):
    return pl.pallas_call(
        _topk_body,
        grid=(T // TB2,),
        in_specs=[
            pl.BlockSpec((TB2, D), lambda i: (i, 0)),
            pl.BlockSpec((TB2, D), lambda i: (i, 0)),
            pl.BlockSpec((M, D), lambda i: (0, 0)),
        ],
        out_specs=[
            pl.BlockSpec((TOPK, TB2), lambda i: (0, i)),
            pl.BlockSpec((TOPK, TB2), lambda i: (0, i)),
        ],
        out_shape=[
            jax.ShapeDtypeStruct((TOPK, T), jnp.int32),
            jax.ShapeDtypeStruct((TOPK, T), jnp.float32),
        ],
    )(h, hwk, memory)


def _attn_body(q_ref, kp_ref, kc_ref, vp_ref, vc_ref, o_ref):
    qi = pl.program_id(1)
    q = q_ref[...].astype(jnp.bfloat16)
    k = jnp.concatenate([kp_ref[...], kc_ref[...]], axis=0).astype(jnp.bfloat16)
    s = lax.dot_general(q, k, (((1,), (1,)), ((), ())),
                        preferred_element_type=jnp.float32)
    s = s * np.float32(1.0 / np.sqrt(DH))
    r = lax.broadcasted_iota(jnp.int32, (TB, 2 * TB), 0)
    c = lax.broadcasted_iota(jnp.int32, (TB, 2 * TB), 1)
    jglob = qi * TB + c - TB
    allowed = (c > r) & (c <= r + TB) & (jglob >= 0)
    s = jnp.where(allowed, s, jnp.float32(-1e30))
    m = jnp.max(s, axis=1, keepdims=True)
    p = jnp.exp(s - m)
    v = jnp.concatenate([vp_ref[...], vc_ref[...]], axis=0).astype(jnp.bfloat16)
    ctx = jnp.dot(p.astype(jnp.bfloat16), v,
                  preferred_element_type=jnp.float32)
    o_ref[...] = ctx / jnp.sum(p, axis=1, keepdims=True)


def _attn(q, k, v):
    def prev_map(hh, qi):
        return (jnp.maximum(qi - 1, 0), hh)

    return pl.pallas_call(
        _attn_body,
        grid=(H, T // TB),
        in_specs=[
            pl.BlockSpec((TB, DH), lambda hh, qi: (qi, hh)),
            pl.BlockSpec((TB, DH), prev_map),
            pl.BlockSpec((TB, DH), lambda hh, qi: (qi, hh)),
            pl.BlockSpec((TB, DH), prev_map),
            pl.BlockSpec((TB, DH), lambda hh, qi: (qi, hh)),
        ],
        out_specs=pl.BlockSpec((TB, DH), lambda hh, qi: (qi, hh)),
        out_shape=jax.ShapeDtypeStruct((T, D), jnp.float32),
    )(q, k, k, v, v)


def _final_body(ctx_ref, oraw_ref, wo_ref, wv_ref, bw_ref, bb_ref, out_ref):
    wo = wo_ref[...].astype(jnp.bfloat16)
    olocal = jnp.dot(ctx_ref[...].astype(jnp.bfloat16), wo,
                     preferred_element_type=jnp.float32)
    z = jnp.sum(olocal * bw_ref[...], axis=1, keepdims=True) + bb_ref[0]
    gate = jax.nn.sigmoid(z)
    vm = jnp.dot(oraw_ref[...].astype(jnp.bfloat16),
                 wv_ref[...].astype(jnp.bfloat16),
                 preferred_element_type=jnp.float32)
    om = jnp.dot(vm.astype(jnp.bfloat16), wo,
                 preferred_element_type=jnp.float32)
    out_ref[...] = olocal + gate * om


def _final(ctx, o_raw, Wo, Wv, bw_row, bb):
    return pl.pallas_call(
        _final_body,
        grid=(T // TB,),
        in_specs=[
            pl.BlockSpec((TB, D), lambda i: (i, 0)),
            pl.BlockSpec((TB, D), lambda i: (i, 0)),
            pl.BlockSpec((D, D), lambda i: (0, 0)),
            pl.BlockSpec((D, D), lambda i: (0, 0)),
            pl.BlockSpec((1, D), lambda i: (0, 0)),
            pl.BlockSpec(memory_space=pltpu.SMEM),
        ],
        out_specs=pl.BlockSpec((TB, D), lambda i: (i, 0)),
        out_shape=jax.ShapeDtypeStruct((T, D), jnp.float32),
    )(ctx, o_raw, Wo, Wv, bw_row, bb)


def _sc_gather_body(mem_hbm, idx_hbm, w_hbm, out_hbm, idx_v, w_v, rows_v,
                    acc_v, sem):
    wid = lax.axis_index("s") * 2 + lax.axis_index("c")
    pltpu.sync_copy(idx_hbm.at[wid], idx_v)
    pltpu.sync_copy(w_hbm.at[wid], w_v)

    def chunk(it, carry):
        pltpu.async_copy(mem_hbm.at[idx_v.at[it]], rows_v, sem).wait()
        wa = w_v[it, pl.ds(0, 16)]
        wb = w_v[it, pl.ds(16, 16)]

        def col(cc, carry2):
            for tt in range(CH):
                a = jnp.zeros((16,), jnp.float32)
                for i in range(TOPK):
                    j = tt * TOPK + i
                    w = wa[j] if j < 16 else wb[j - 16]
                    a = a + w * rows_v[j, pl.ds(cc * 16, 16)]
                acc_v[tt, pl.ds(cc * 16, 16)] = a
            return carry2

        lax.fori_loop(0, D // 16, col, 0)
        pltpu.sync_copy(acc_v,
                        out_hbm.at[pl.ds(wid * TOK_PER_W + it * CH, CH)])
        return carry

    lax.fori_loop(0, NCH, chunk, 0)


def _sc_gather(memory, idx_w, w_w):
    fn = functools.partial(
        pl.kernel,
        mesh=plsc.VectorSubcoreMesh(core_axis_name="c", subcore_axis_name="s"),
        out_type=jax.ShapeDtypeStruct((T, D), jnp.float32),
        scratch_types=[
            pltpu.VMEM((NCH, ROWS), jnp.int32),
            pltpu.VMEM((NCH, ROWS), jnp.float32),
            pltpu.VMEM((ROWS, D), jnp.float32),
            pltpu.VMEM((CH, D), jnp.float32),
            pltpu.SemaphoreType.DMA,
        ],
    )(_sc_gather_body)
    return fn(memory, idx_w, w_w)


def kernel(hidden_states, Wq, Wk, Wv, Wo, bypass_w, bypass_b, memory):
    b, t, d = hidden_states.shape
    h = hidden_states.reshape(t, d)
    q, k, v, hwk = _proj(h, Wq, Wk, Wv)
    idx8, mw8 = _topk(h, hwk, memory)
    ctx = _attn(q, k, v)
    idx_w = idx8.T.reshape(NW, NCH, ROWS)
    w_w = mw8.T.reshape(NW, NCH, ROWS)
    o_raw = _sc_gather(memory, idx_w, w_w)
    out = _final(ctx, o_raw, Wo, Wv, bypass_w.reshape(1, d), bypass_b)
    return out.reshape(b, t, d)


# speedup vs baseline: 1.2856x; 1.2856x over previous
import functools

import jax
import jax.numpy as jnp
import numpy as np
from jax import lax
from jax.experimental import pallas as pl
from jax.experimental.pallas import tpu as pltpu
from jax.experimental.pallas import tpu_sc as plsc

T, D, H, DH = 2048, 2048, 16, 128
M = 4096
WINDOW = 256
TOPK = 8
TB = 256
NW = 32
TOK_PER_W = T // NW
CH = 2
NCH = TOK_PER_W // CH
ROWS = CH * TOPK


def _proj_body(h_ref, wq_ref, wk_ref, wv_ref, q_ref, k_ref, v_ref):
    h = h_ref[...]
    q_ref[...] = jnp.dot(h, wq_ref[...],
                         preferred_element_type=jnp.float32).astype(jnp.bfloat16)
    k_ref[...] = jnp.dot(h, wk_ref[...],
                         preferred_element_type=jnp.float32).astype(jnp.bfloat16)
    v_ref[...] = jnp.dot(h, wv_ref[...],
                         preferred_element_type=jnp.float32).astype(jnp.bfloat16)


def _proj(h_bf, Wq_bf, Wk_bf, Wv_bf):
    out = jax.ShapeDtypeStruct((T, D), jnp.bfloat16)
    return pl.pallas_call(
        _proj_body,
        grid=(D // TB,),
        in_specs=[
            pl.BlockSpec((T, D), lambda j: (0, 0)),
            pl.BlockSpec((D, TB), lambda j: (0, j)),
            pl.BlockSpec((D, TB), lambda j: (0, j)),
            pl.BlockSpec((D, TB), lambda j: (0, j)),
        ],
        out_specs=[pl.BlockSpec((T, TB), lambda j: (0, j))] * 3,
        out_shape=[out, out, out],
    )(h_bf, Wq_bf, Wk_bf, Wv_bf)


TB2 = 128


def _topk_body(h_ref, wkt_ref, mem_ref, idx_ref, mw_ref):
    h = h_ref[...]
    hwk = lax.dot_general(h, wkt_ref[...], (((1,), (1,)), ((), ())),
                          preferred_element_type=jnp.float32)
    hh = jnp.concatenate([h, hwk.astype(jnp.bfloat16)], axis=0)
    sml = lax.dot_general(hh, mem_ref[...], (((1,), (1,)), ((), ())),
                          preferred_element_type=jnp.float32)
    s = sml[:TB2, :]
    ml = sml[TB2:, :]
    colid = lax.broadcasted_iota(jnp.int32, (TB2, M), 1)
    mls = []
    for i in range(TOPK):
        mx = jnp.max(s, axis=1, keepdims=True)
        amin = jnp.min(jnp.where(s >= mx, colid, M), axis=1, keepdims=True)
        onehot = colid == amin
        mls.append(jnp.sum(jnp.where(onehot, ml, 0.0), axis=1))
        idx_ref[i, :] = amin[:, 0]
        s = jnp.where(onehot, -jnp.inf, s)
    mlk = jnp.stack(mls, axis=0) * np.float32(1.0 / np.sqrt(D))
    z = mlk - jnp.max(mlk, axis=0, keepdims=True)
    e = jnp.exp(z)
    mw_ref[...] = e / jnp.sum(e, axis=0, keepdims=True)


def _topk(h_bf, Wk_bf, mem_bf):
    return pl.pallas_call(
        _topk_body,
        grid=(T // TB2,),
        in_specs=[
            pl.BlockSpec((TB2, D), lambda i: (i, 0)),
            pl.BlockSpec((D, D), lambda i: (0, 0)),
            pl.BlockSpec((M, D), lambda i: (0, 0)),
        ],
        out_specs=[
            pl.BlockSpec((TOPK, TB2), lambda i: (0, i)),
            pl.BlockSpec((TOPK, TB2), lambda i: (0, i)),
        ],
        out_shape=[
            jax.ShapeDtypeStruct((TOPK, T), jnp.int32),
            jax.ShapeDtypeStruct((TOPK, T), jnp.float32),
        ],
    )(h_bf, Wk_bf, mem_bf)


def _attn_body(q_ref, kp_ref, kc_ref, vp_ref, vc_ref, o_ref):
    qi = pl.program_id(1)
    q = q_ref[...]
    k = jnp.concatenate([kp_ref[...], kc_ref[...]], axis=0)
    s = lax.dot_general(q, k, (((1,), (1,)), ((), ())),
                        preferred_element_type=jnp.float32)
    s = s * np.float32(1.0 / np.sqrt(DH))
    r = lax.broadcasted_iota(jnp.int32, (TB, 2 * TB), 0)
    c = lax.broadcasted_iota(jnp.int32, (TB, 2 * TB), 1)
    jglob = qi * TB + c - TB
    allowed = (c > r) & (c <= r + TB) & (jglob >= 0)
    s = jnp.where(allowed, s, jnp.float32(-1e30))
    m = jnp.max(s, axis=1, keepdims=True)
    p = jnp.exp(s - m)
    v = jnp.concatenate([vp_ref[...], vc_ref[...]], axis=0)
    ctx = jnp.dot(p.astype(jnp.bfloat16), v,
                  preferred_element_type=jnp.float32)
    o_ref[...] = (ctx / jnp.sum(p, axis=1, keepdims=True)).astype(jnp.bfloat16)


def _attn(q, k, v):
    def prev_map(hh, qi):
        return (jnp.maximum(qi - 1, 0), hh)

    return pl.pallas_call(
        _attn_body,
        grid=(H, T // TB),
        in_specs=[
            pl.BlockSpec((TB, DH), lambda hh, qi: (qi, hh)),
            pl.BlockSpec((TB, DH), prev_map),
            pl.BlockSpec((TB, DH), lambda hh, qi: (qi, hh)),
            pl.BlockSpec((TB, DH), prev_map),
            pl.BlockSpec((TB, DH), lambda hh, qi: (qi, hh)),
        ],
        out_specs=pl.BlockSpec((TB, DH), lambda hh, qi: (qi, hh)),
        out_shape=jax.ShapeDtypeStruct((T, D), jnp.bfloat16),
    )(q, k, k, v, v)


def _final_body(ctx_ref, oraw_ref, wo_ref, wv_ref, bw_ref, bb_ref, out_ref):
    wo = wo_ref[...]
    olocal = jnp.dot(ctx_ref[...], wo, preferred_element_type=jnp.float32)
    z = jnp.sum(olocal * bw_ref[...], axis=1, keepdims=True) + bb_ref[0]
    gate = jax.nn.sigmoid(z)
    vm = jnp.dot(oraw_ref[...].astype(jnp.bfloat16), wv_ref[...],
                 preferred_element_type=jnp.float32)
    om = jnp.dot(vm.astype(jnp.bfloat16), wo,
                 preferred_element_type=jnp.float32)
    out_ref[...] = olocal + gate * om


def _final(ctx, o_raw, Wo_bf, Wv_bf, bw_row, bb):
    return pl.pallas_call(
        _final_body,
        grid=(T // TB,),
        in_specs=[
            pl.BlockSpec((TB, D), lambda i: (i, 0)),
            pl.BlockSpec((TB, D), lambda i: (i, 0)),
            pl.BlockSpec((D, D), lambda i: (0, 0)),
            pl.BlockSpec((D, D), lambda i: (0, 0)),
            pl.BlockSpec((1, D), lambda i: (0, 0)),
            pl.BlockSpec(memory_space=pltpu.SMEM),
        ],
        out_specs=pl.BlockSpec((TB, D), lambda i: (i, 0)),
        out_shape=jax.ShapeDtypeStruct((T, D), jnp.float32),
    )(ctx, o_raw, Wo_bf, Wv_bf, bw_row, bb)


def _sc_compute_chunk(rows_v, w, acc_v, out_hbm, row0):
    def col(cc, carry):
        for tt in range(CH):
            a = jnp.zeros((16,), jnp.float32)
            for i in range(TOPK):
                j = tt * TOPK + i
                a = a + w[j] * rows_v[j, pl.ds(cc * 16, 16)]
            acc_v[tt, pl.ds(cc * 16, 16)] = a
        return carry

    lax.fori_loop(0, D // 16, col, 0, unroll=4)
    pltpu.sync_copy(acc_v, out_hbm.at[pl.ds(row0, CH)])


def _sc_gather_body(mem_hbm, idx_hbm, w_hbm, out_hbm, idx_v, w_v, rows0_v,
                    rows1_v, acc_v, sem0, sem1):
    wid = lax.axis_index("s") * 2 + lax.axis_index("c")
    pltpu.sync_copy(idx_hbm.at[wid], idx_v)
    pltpu.sync_copy(w_hbm.at[wid], w_v)
    base = wid * TOK_PER_W
    pltpu.async_copy(mem_hbm.at[idx_v.at[0]], rows0_v, sem0)

    def pair(p, carry):
        c0 = 2 * p
        pltpu.async_copy(mem_hbm.at[idx_v.at[c0 + 1]], rows1_v, sem1)
        pltpu.make_async_copy(mem_hbm.at[idx_v.at[c0]], rows0_v, sem0).wait()
        _sc_compute_chunk(rows0_v, w_v[c0, :], acc_v, out_hbm,
                          base + c0 * CH)

        @pl.when(p + 1 < NCH // 2)
        def _():
            pltpu.async_copy(mem_hbm.at[idx_v.at[c0 + 2]], rows0_v, sem0)

        pltpu.make_async_copy(mem_hbm.at[idx_v.at[c0 + 1]], rows1_v,
                              sem1).wait()
        _sc_compute_chunk(rows1_v, w_v[c0 + 1, :], acc_v, out_hbm,
                          base + (c0 + 1) * CH)
        return carry

    lax.fori_loop(0, NCH // 2, pair, 0)


def _sc_gather(memory, idx_w, w_w):
    fn = functools.partial(
        pl.kernel,
        mesh=plsc.VectorSubcoreMesh(core_axis_name="c", subcore_axis_name="s"),
        out_type=jax.ShapeDtypeStruct((T, D), jnp.float32),
        scratch_types=[
            pltpu.VMEM((NCH, ROWS), jnp.int32),
            pltpu.VMEM((NCH, ROWS), jnp.float32),
            pltpu.VMEM((ROWS, D), jnp.float32),
            pltpu.VMEM((ROWS, D), jnp.float32),
            pltpu.VMEM((CH, D), jnp.float32),
            pltpu.SemaphoreType.DMA,
            pltpu.SemaphoreType.DMA,
        ],
    )(_sc_gather_body)
    return fn(memory, idx_w, w_w)


def kernel(hidden_states, Wq, Wk, Wv, Wo, bypass_w, bypass_b, memory):
    b, t, d = hidden_states.shape
    h = hidden_states.reshape(t, d)
    h_bf = h.astype(jnp.bfloat16)
    Wq_bf = Wq.astype(jnp.bfloat16)
    Wk_bf = Wk.astype(jnp.bfloat16)
    Wv_bf = Wv.astype(jnp.bfloat16)
    Wo_bf = Wo.astype(jnp.bfloat16)
    mem_bf = memory.astype(jnp.bfloat16)
    idx8, mw8 = _topk(h_bf, Wk_bf, mem_bf)
    idx_w = idx8.T.reshape(NW, NCH, ROWS)
    w_w = mw8.T.reshape(NW, NCH, ROWS)
    o_raw = _sc_gather(memory, idx_w, w_w)
    q, k, v = _proj(h_bf, Wq_bf, Wk_bf, Wv_bf)
    ctx = _attn(q, k, v)
    out = _final(ctx, o_raw, Wo_bf, Wv_bf, bypass_w.reshape(1, d), bypass_b)
    return out.reshape(b, t, d)
